# baseline (speedup 1.0000x reference)
import functools
import math

import jax
import jax.numpy as jnp
from jax import lax
from jax.experimental import pallas as pl
from jax.experimental.pallas import tpu as pltpu

N_HIDDEN = 256
N_EDGE_HIDDEN = 128
MSG_NORM = 100.0
_EPS = 1e-5


def _silu(x):
    return x * (1.0 / (1.0 + jnp.exp(-x)))


def _layernorm(x, g, b):
    m = jnp.mean(x, axis=-1, keepdims=True)
    v = jnp.mean((x - m) ** 2, axis=-1, keepdims=True)
    return (x - m) * jax.lax.rsqrt(v + _EPS) * g + b


def _row_block(bs, d):
    return pl.BlockSpec((bs, d), lambda i: (i, 0))


def _full_block(shape):
    nd = len(shape)
    return pl.BlockSpec(shape, lambda i: (0,) * nd)


def _node_embed_body(x_ref, r_ref, w1_ref, b1_ref, w2_ref, b2_ref,
                     lng_ref, lnb_ref, rw1_ref, rb1_ref, rw2_ref, rb2_ref,
                     gate_ref, o_ref):
    x = x_ref[...]
    h = _silu(jnp.dot(x, w1_ref[...]) + b1_ref[...])
    h = _silu(jnp.dot(h, w2_ref[...]) + b2_ref[...])
    h = _layernorm(h, lng_ref[...], lnb_ref[...])
    r = _silu(jnp.dot(r_ref[...], rw1_ref[...]) + rb1_ref[...])
    r = jnp.dot(r, rw2_ref[...]) + rb2_ref[...]
    o_ref[...] = h + gate_ref[0, 0] * r


def _node_embed(x, r, se, re, bs):
    n, din = x.shape
    dr = r.shape[1]
    d = N_HIDDEN
    return pl.pallas_call(
        _node_embed_body,
        grid=(n // bs,),
        in_specs=[
            _row_block(bs, din), _row_block(bs, dr),
            _full_block((din, d)), _full_block((1, d)),
            _full_block((d, d)), _full_block((1, d)),
            _full_block((1, d)), _full_block((1, d)),
            _full_block((dr, d)), _full_block((1, d)),
            _full_block((d, d)), _full_block((1, d)),
            _full_block((1, 1)),
        ],
        out_specs=_row_block(bs, d),
        out_shape=jax.ShapeDtypeStruct((n, d), jnp.float32),
    )(x, r,
      se["l1"]["W"], se["l1"]["b"].reshape(1, -1),
      se["l2"]["W"], se["l2"]["b"].reshape(1, -1),
      se["ln"]["g"].reshape(1, -1), se["ln"]["b"].reshape(1, -1),
      re["l1"]["W"], re["l1"]["b"].reshape(1, -1),
      re["l2"]["W"], re["l2"]["b"].reshape(1, -1),
      re["gate"].reshape(1, 1))


def _mlp_ln_body(x_ref, w1_ref, b1_ref, w2_ref, b2_ref, lng_ref, lnb_ref,
                 o_ref):
    h = _silu(jnp.dot(x_ref[...], w1_ref[...]) + b1_ref[...])
    h = _silu(jnp.dot(h, w2_ref[...]) + b2_ref[...])
    o_ref[...] = _layernorm(h, lng_ref[...], lnb_ref[...])


def _mlp_ln(x, p, bs):
    n, din = x.shape
    d = p["l1"]["W"].shape[1]
    return pl.pallas_call(
        _mlp_ln_body,
        grid=(n // bs,),
        in_specs=[
            _row_block(bs, din),
            _full_block((din, d)), _full_block((1, d)),
            _full_block((d, d)), _full_block((1, d)),
            _full_block((1, d)), _full_block((1, d)),
        ],
        out_specs=_row_block(bs, d),
        out_shape=jax.ShapeDtypeStruct((n, d), jnp.float32),
    )(x, p["l1"]["W"], p["l1"]["b"].reshape(1, -1),
      p["l2"]["W"], p["l2"]["b"].reshape(1, -1),
      p["ln"]["g"].reshape(1, -1), p["ln"]["b"].reshape(1, -1))


def _msg_body(hs_ref, e_ref, w1a_ref, w1b_ref, b1_ref, w2_ref, b2_ref,
              o_ref):
    h = jnp.dot(hs_ref[...], w1a_ref[...])
    h += jnp.dot(e_ref[...], w1b_ref[...])
    h = _silu(h + b1_ref[...])
    o_ref[...] = _silu(jnp.dot(h, w2_ref[...]) + b2_ref[...])


def _msg_mlp(hs, e, cp, bs):
    n = hs.shape[0]
    d = N_HIDDEN
    de = N_EDGE_HIDDEN
    w1 = cp["msg1"]["W"]
    return pl.pallas_call(
        _msg_body,
        grid=(n // bs,),
        in_specs=[
            _row_block(bs, d), _row_block(bs, de),
            _full_block((d, d)), _full_block((de, d)), _full_block((1, d)),
            _full_block((d, d)), _full_block((1, d)),
        ],
        out_specs=_row_block(bs, d),
        out_shape=jax.ShapeDtypeStruct((n, d), jnp.float32),
    )(hs, e, w1[:d], w1[d:], cp["msg1"]["b"].reshape(1, -1),
      cp["msg2"]["W"], cp["msg2"]["b"].reshape(1, -1))


def _node_update_body(h_ref, agg_ref, ln1g_ref, ln1b_ref, w1_ref, b1_ref,
                      w2_ref, b2_ref, ln2g_ref, ln2b_ref, o_ref):
    h = _layernorm(h_ref[...] + agg_ref[...] * (1.0 / MSG_NORM),
                   ln1g_ref[...], ln1b_ref[...])
    r = _silu(jnp.dot(h, w1_ref[...]) + b1_ref[...])
    r = _silu(jnp.dot(r, w2_ref[...]) + b2_ref[...])
    o_ref[...] = _layernorm(h + r, ln2g_ref[...], ln2b_ref[...])


def _node_update(h, agg, cp, bs):
    n = h.shape[0]
    d = N_HIDDEN
    return pl.pallas_call(
        _node_update_body,
        grid=(n // bs,),
        in_specs=[
            _row_block(bs, d), _row_block(bs, d),
            _full_block((1, d)), _full_block((1, d)),
            _full_block((d, d)), _full_block((1, d)),
            _full_block((d, d)), _full_block((1, d)),
            _full_block((1, d)), _full_block((1, d)),
        ],
        out_specs=_row_block(bs, d),
        out_shape=jax.ShapeDtypeStruct((n, d), jnp.float32),
    )(h, agg,
      cp["ln1"]["g"].reshape(1, -1), cp["ln1"]["b"].reshape(1, -1),
      cp["upd1"]["W"], cp["upd1"]["b"].reshape(1, -1),
      cp["upd2"]["W"], cp["upd2"]["b"].reshape(1, -1),
      cp["ln2"]["g"].reshape(1, -1), cp["ln2"]["b"].reshape(1, -1))


def _edge_update_body(hs_ref, hd_ref, e_ref, w1a_ref, w1b_ref, w1c_ref,
                      b1_ref, w2_ref, b2_ref, lng_ref, lnb_ref, o_ref):
    h = jnp.dot(hs_ref[...], w1a_ref[...])
    h += jnp.dot(hd_ref[...], w1b_ref[...])
    h += jnp.dot(e_ref[...], w1c_ref[...])
    h = _silu(h + b1_ref[...])
    eo = _silu(jnp.dot(h, w2_ref[...]) + b2_ref[...])
    o_ref[...] = _layernorm(e_ref[...] + eo, lng_ref[...], lnb_ref[...])


def _edge_update(hs, hd, e, ep, bs):
    n = hs.shape[0]
    d = N_HIDDEN
    de = N_EDGE_HIDDEN
    w1 = ep["l1"]["W"]
    return pl.pallas_call(
        _edge_update_body,
        grid=(n // bs,),
        in_specs=[
            _row_block(bs, d), _row_block(bs, d), _row_block(bs, de),
            _full_block((d, de)), _full_block((d, de)),
            _full_block((de, de)), _full_block((1, de)),
            _full_block((de, de)), _full_block((1, de)),
            _full_block((1, de)), _full_block((1, de)),
        ],
        out_specs=_row_block(bs, de),
        out_shape=jax.ShapeDtypeStruct((n, de), jnp.float32),
    )(hs, hd, e, w1[:d], w1[d:2 * d], w1[2 * d:],
      ep["l1"]["b"].reshape(1, -1),
      ep["l2"]["W"], ep["l2"]["b"].reshape(1, -1),
      ep["ln"]["g"].reshape(1, -1), ep["ln"]["b"].reshape(1, -1))


def _head_body(x_ref, w1_ref, b1_ref, w2_ref, b2_ref, o_ref):
    h = _silu(jnp.dot(x_ref[...], w1_ref[...]) + b1_ref[...])
    o_ref[...] = jnp.dot(h, w2_ref[...]) + b2_ref[...]


def _head(x, hp, dout, bs):
    n, din = x.shape
    d = hp["l1"]["W"].shape[1]
    dp = 128
    w2 = jnp.pad(hp["l2"]["W"], ((0, 0), (0, dp - dout)))
    b2 = jnp.pad(hp["l2"]["b"], (0, dp - dout)).reshape(1, -1)
    out = pl.pallas_call(
        _head_body,
        grid=(n // bs,),
        in_specs=[
            _row_block(bs, din),
            _full_block((din, d)), _full_block((1, d)),
            _full_block((d, dp)), _full_block((1, dp)),
        ],
        out_specs=_row_block(bs, dp),
        out_shape=jax.ShapeDtypeStruct((n, dp), jnp.float32),
    )(x, hp["l1"]["W"], hp["l1"]["b"].reshape(1, -1), w2, b2)
    return out[:, :dout]


def _gather_rows(table, idx):
    return jnp.take(table, idx, axis=0)


def _segment_sum(values, idx, n):
    return jax.ops.segment_sum(values, idx, num_segments=n)


def kernel(node_feats, edge_feats, random_feats, params, edge_index):
    src = edge_index[0]
    dst = edge_index[1]
    n = node_feats.shape[0]
    n_edges = edge_feats.shape[0]
    bs_n = 2000
    bs_e = 2000

    h = _node_embed(node_feats, random_feats,
                    params["scalar_emb"], params["rand_emb"], bs_n)
    e = _mlp_ln(edge_feats, params["edge_emb"], bs_e)

    conv_idx = 0
    for _update in range(2):
        for _c in range(2):
            cp = params["convs"][conv_idx]
            conv_idx += 1
            hs = _gather_rows(h, src)
            msg = _msg_mlp(hs, e, cp, bs_e)
            agg = _segment_sum(msg, dst, n)
            h = _node_update(h, agg, cp, bs_n)
        ep = params["edge_upd"]
        hs = _gather_rows(h, src)
        hd = _gather_rows(h, dst)
        e = _edge_update(hs, hd, e, ep, bs_e)

    a_logits = _head(h, params["head_a"], 16, bs_n)
    c_logits = _head(h, params["head_c"], 6, bs_n)
    e_logits = _head(e, params["head_e"], 5, bs_e)
    return (a_logits, c_logits, e_logits)
